# Initial kernel scaffold; baseline (speedup 1.0000x reference)
import jax
import jax.numpy as jnp
from jax.experimental import pallas as pl


def kernel(h, edges, distances, W_edg1, b_edg1, W_edg2, b_edg2, W_edgi, b_edgi, W_node1, b_node1, W_node2, b_node2):
    raise NotImplementedError("write your pallas kernel here")


# speedup vs baseline: 5.4410x; 5.4410x over previous
import functools

import jax
import jax.numpy as jnp
from jax import lax
from jax.experimental import pallas as pl
from jax.experimental.pallas import tpu as pltpu
from jax.experimental.pallas import tpu_sc as plsc

_NC, _NS, _L = 2, 16, 16
_NW = _NC * _NS
_NPAD = 10240
_PPT = _NPAD // _NS
_CW = 128
_CH = 80
_EPW = _CH * _CW
_EPAD = _NW * _EPW


def _sc_segment_sum(row_p, dist_p):
    mesh = plsc.VectorSubcoreMesh(core_axis_name="c", subcore_axis_name="s")

    @functools.partial(
        pl.kernel,
        out_type=jax.ShapeDtypeStruct((_NC, _NPAD), jnp.float32),
        mesh=mesh,
        scratch_types=[
            pltpu.VMEM((_CH, _CW), jnp.int32),
            pltpu.VMEM((_CH, _CW), jnp.float32),
            pltpu.VMEM((_PPT,), jnp.float32),
            pltpu.VMEM_SHARED((_NPAD,), jnp.float32),
            pltpu.SemaphoreType.DMA,
        ],
    )
    def seg_sum(row_hbm, dist_hbm, out_hbm, idx_v, val_v, zero_v, agg_sh, sem):
        c = lax.axis_index("c")
        s = lax.axis_index("s")
        wid = c * _NS + s
        for i in range(_PPT // _L):
            zero_v[pl.ds(i * _L, _L)] = jnp.zeros((_L,), jnp.float32)
        pltpu.sync_copy(zero_v, agg_sh.at[pl.ds(s * _PPT, _PPT)])
        pltpu.sync_copy(row_hbm.at[wid], idx_v)
        pltpu.sync_copy(dist_hbm.at[wid], val_v)
        plsc.subcore_barrier()
        copies = [
            pltpu.async_copy(val_v.at[j], agg_sh.at[idx_v.at[j]], sem, add=True)
            for j in range(_CH)
        ]
        for cp in copies:
            cp.wait()
        plsc.subcore_barrier()
        pltpu.sync_copy(agg_sh.at[pl.ds(s * _PPT, _PPT)],
                        out_hbm.at[c, pl.ds(s * _PPT, _PPT)])

    return seg_sum(row_p, dist_p)


_BR = 1000


def _tc_node_mlp(h, aggcol, w1a, w1b, b1, w2, b2):
    def body(h_ref, a_ref, w1a_ref, w1b_ref, b1_ref, w2_ref, b2_ref, o_ref):
        x = h_ref[...]
        pre = (jnp.dot(x, w1a_ref[...], preferred_element_type=jnp.float32)
               + a_ref[...] * w1b_ref[...] + b1_ref[...])
        t = pre * jax.nn.sigmoid(pre)
        o_ref[...] = (x + jnp.dot(t, w2_ref[...], preferred_element_type=jnp.float32)
                      + b2_ref[...])

    n = h.shape[0]
    return pl.pallas_call(
        body,
        grid=(n // _BR,),
        in_specs=[
            pl.BlockSpec((_BR, 128), lambda i: (i, 0)),
            pl.BlockSpec((_BR, 1), lambda i: (i, 0)),
            pl.BlockSpec((128, 128), lambda i: (0, 0)),
            pl.BlockSpec((1, 128), lambda i: (0, 0)),
            pl.BlockSpec((1, 128), lambda i: (0, 0)),
            pl.BlockSpec((128, 128), lambda i: (0, 0)),
            pl.BlockSpec((1, 128), lambda i: (0, 0)),
        ],
        out_specs=pl.BlockSpec((_BR, 128), lambda i: (i, 0)),
        out_shape=jax.ShapeDtypeStruct((n, 128), jnp.float32),
    )(h, aggcol, w1a, w1b, b1, w2, b2)


def kernel(h, edges, distances, W_edg1, b_edg1, W_edg2, b_edg2, W_edgi, b_edgi,
           W_node1, b_node1, W_node2, b_node2):
    n = h.shape[0]
    row = edges[0].astype(jnp.int32)
    dist = distances.reshape(-1).astype(jnp.float32)
    pad = _EPAD - row.shape[0]
    row_p = jnp.concatenate(
        [row, jnp.full((pad,), _NPAD - 1, jnp.int32)]).reshape(_NW, _CH, _CW)
    dist_p = jnp.concatenate(
        [dist, jnp.zeros((pad,), jnp.float32)]).reshape(_NW, _CH, _CW)

    parts = _sc_segment_sum(row_p, dist_p)
    aggcol = (parts[0, :n] + parts[1, :n]).reshape(n, 1)

    w1a = W_node1[:128]
    w1b = (W_node1[128] / 100.0).reshape(1, 128)
    b1 = b_node1.reshape(1, 128)
    b2 = b_node2.reshape(1, 128)
    return _tc_node_mlp(h, aggcol, w1a, w1b, b1, W_node2, b2)
